# Initial kernel scaffold; baseline (speedup 1.0000x reference)
import jax
import jax.numpy as jnp
from jax.experimental import pallas as pl


def kernel(logits, targets):
    raise NotImplementedError("write your pallas kernel here")


# speedup vs baseline: 1.4361x; 1.4361x over previous
import functools

import jax
import jax.numpy as jnp
from jax import lax
from jax.experimental import pallas as pl
from jax.experimental.pallas import tpu as pltpu
from jax.experimental.pallas import tpu_sc as plsc

N_CATS = 10
B0, B1 = 4, 8192
TOK = B0 * B1

_INFO = plsc.get_sparse_core_info()
NC = _INFO.num_cores
NS = _INFO.num_subcores
L = _INFO.num_lanes
NW = NC * NS
TPW = TOK // NW
GROUPS = TPW // L


def _sc_body(lg_hbm, tg_hbm, out_hbm, lg_v, tg_v, out_v):
    wid = lax.axis_index("s") * NC + lax.axis_index("c")
    tok0 = wid * TPW
    pltpu.sync_copy(lg_hbm.at[pl.ds(tok0 * N_CATS, TPW * N_CATS)], lg_v)
    pltpu.sync_copy(tg_hbm.at[pl.ds(tok0, TPW)], tg_v)

    lane = lax.iota(jnp.int32, L)
    lane10 = lane * N_CATS
    zf = jnp.zeros((L,), jnp.float32)

    def body(j, carry):
        accN, accB, acc1t, acc2t, acc1p, acc2p = carry
        base = j * (L * N_CATS)
        amax = plsc.load_gather(lg_v, [lane10 + base])
        aidx = jnp.zeros((L,), jnp.int32)
        for r in range(1, N_CATS):
            v = plsc.load_gather(lg_v, [lane10 + (base + r)])
            gt = v > amax
            aidx = jnp.where(gt, jnp.int32(r), aidx)
            amax = jnp.where(gt, v, amax)
        t = tg_v[pl.ds(j * L, L)]
        m = jnp.where(t > 0, 1.0, 0.0)
        tf = t.astype(jnp.float32)
        pf = aidx.astype(jnp.float32)
        d = tf - pf
        return (accN + m,
                accB + m * d * d,
                acc1t + m * tf,
                acc2t + m * tf * tf,
                acc1p + m * pf,
                acc2p + m * pf * pf)

    accN, accB, acc1t, acc2t, acc1p, acc2p = lax.fori_loop(
        0, GROUPS, body, (zf, zf, zf, zf, zf, zf))

    sums = (jnp.sum(accN), jnp.sum(accB), jnp.sum(acc1t),
            jnp.sum(acc2t), jnp.sum(acc1p), jnp.sum(acc2p))
    out16 = zf
    for k, s in enumerate(sums):
        out16 = out16 + jnp.where(lane == k, s, 0.0)
    out_v[...] = out16
    pltpu.sync_copy(out_v, out_hbm.at[wid])


_sc_partials = functools.partial(
    pl.kernel,
    mesh=plsc.VectorSubcoreMesh(core_axis_name="c", subcore_axis_name="s"),
    out_type=jax.ShapeDtypeStruct((NW, L), jnp.float32),
    scratch_types=[
        pltpu.VMEM((TPW * N_CATS,), jnp.float32),
        pltpu.VMEM((TPW,), jnp.int32),
        pltpu.VMEM((L,), jnp.float32),
    ],
    compiler_params=pltpu.CompilerParams(needs_layout_passes=False),
)(_sc_body)


def _finalize_body(p_ref, o_ref):
    x = p_ref[...]
    col = lax.broadcasted_iota(jnp.int32, (NW, L), 1)

    def s(k):
        return jnp.sum(jnp.where(col == k, x, 0.0))

    N, B, S1t, S2t, S1p, S2p = s(0), s(1), s(2), s(3), s(4), s(5)
    wsq = jnp.float32((N_CATS - 1) ** 2)
    num = N - B / wsq
    den = N * N - (N * S2t - 2.0 * S1t * S1p + N * S2p) / wsq
    qwk = jnp.where(den == 0.0, 0.0, N * num / den)
    loss = jnp.where(N == 0.0, 0.0, 1.0 - qwk)
    o_ref[...] = jnp.reshape(loss, (1, 1))


def kernel(logits, targets):
    lg = logits.reshape(-1).astype(jnp.float32)
    tg = targets.reshape(-1).astype(jnp.int32)
    partials = _sc_partials(lg, tg)
    loss2d = pl.pallas_call(
        _finalize_body,
        out_shape=jax.ShapeDtypeStruct((1, 1), jnp.float32),
    )(partials)
    return loss2d[0, 0]
